# Initial kernel scaffold; baseline (speedup 1.0000x reference)
import functools

import jax
import jax.numpy as jnp
from jax import lax
from jax.experimental import pallas as pl
from jax.experimental.pallas import tpu as pltpu
from jax.experimental.pallas import tpu_sc as plsc

VNUM = 10000
E = 320000
D = 128
EPS = 1e-7

NC = 2
NS = 16
L = 16
NW = NC * NS

CHUNK = 128
CPW = -(-E // (NW * CHUNK))
ROWS = NW * CPW
EP = ROWS * CHUNK

RPS = VNUM // NS
ZCH = 125

_mesh = plsc.VectorSubcoreMesh(core_axis_name="c", subcore_axis_name="s")


@functools.partial(
    pl.kernel,
    out_type=jax.ShapeDtypeStruct((NC, VNUM, D), jnp.float32),
    mesh=_mesh,
    scratch_types=[
        pltpu.VMEM_SHARED((VNUM, D), jnp.float32),
        pltpu.VMEM((CHUNK,), jnp.int32),
        pltpu.VMEM((CHUNK,), jnp.int32),
        pltpu.VMEM((CHUNK,), jnp.float32),
        pltpu.VMEM((CHUNK,), jnp.float32),
        pltpu.VMEM((CHUNK, D), jnp.float32),
    ],
)
def _sc_graphconv(sidx_hbm, tidx_hbm, enorm_hbm, esgn_hbm, vrepr_hbm,
                  out_hbm, acc, si_v, ti_v, w_v, e_v, rows_v):
    c = lax.axis_index("c")
    s = lax.axis_index("s")
    wid = s * NC + c

    @pl.loop(0, ZCH)
    def _(i):
        @pl.loop(0, D, step=L)
        def _(j):
            rows_v[i, pl.ds(j, L)] = jnp.zeros((L,), jnp.float32)

    @pl.loop(0, RPS // ZCH)
    def _(k):
        pltpu.sync_copy(rows_v.at[pl.ds(0, ZCH)],
                        acc.at[pl.ds(s * RPS + k * ZCH, ZCH)])

    plsc.subcore_barrier()

    @pl.loop(0, CPW)
    def _(i):
        row = wid * CPW + i
        pltpu.sync_copy(sidx_hbm.at[row], si_v)
        pltpu.sync_copy(tidx_hbm.at[row], ti_v)
        pltpu.sync_copy(enorm_hbm.at[row], w_v)
        pltpu.sync_copy(esgn_hbm.at[row], e_v)
        pltpu.sync_copy(vrepr_hbm.at[si_v], rows_v)

        @pl.loop(0, CHUNK, step=L)
        def _(j):
            w_v[pl.ds(j, L)] = w_v[pl.ds(j, L)] * e_v[pl.ds(j, L)]

        @pl.loop(0, CHUNK)
        def _(e):
            wv = plsc.load_gather(w_v, [jnp.full((L,), e, jnp.int32)])

            @pl.loop(0, D, step=L)
            def _(j):
                rows_v[e, pl.ds(j, L)] = rows_v[e, pl.ds(j, L)] * wv

        pltpu.sync_copy(rows_v, acc.at[ti_v], add=True)

    plsc.subcore_barrier()

    @pl.loop(0, RPS // ZCH)
    def _(k):
        r0 = s * RPS + k * ZCH
        pltpu.sync_copy(acc.at[pl.ds(r0, ZCH)], out_hbm.at[c, pl.ds(r0, ZCH)])


def _tc_body(p_ref, lw_ref, lb_ref, sw_ref, sb_ref, loc_ref, std_ref):
    ptr = p_ref[0] + p_ref[1]
    dn = (((1,), (1,)), ((), ()))
    loc = lax.dot_general(ptr, lw_ref[...], dn,
                          preferred_element_type=jnp.float32,
                          precision=lax.Precision.HIGHEST)
    loc_ref[...] = loc + lb_ref[...]
    pre = lax.dot_general(ptr, sw_ref[...], dn,
                          preferred_element_type=jnp.float32,
                          precision=lax.Precision.HIGHEST)
    std_ref[...] = jax.nn.softplus(pre + sb_ref[...]) + EPS


_TCB = 1000


def _tc_project(partials, loc_W, loc_b, std_W, std_b):
    grid = (VNUM // _TCB,)
    return pl.pallas_call(
        _tc_body,
        grid=grid,
        in_specs=[
            pl.BlockSpec((NC, _TCB, D), lambda i: (0, i, 0)),
            pl.BlockSpec((D, D), lambda i: (0, 0)),
            pl.BlockSpec((1, D), lambda i: (0, 0)),
            pl.BlockSpec((D, D), lambda i: (0, 0)),
            pl.BlockSpec((1, D), lambda i: (0, 0)),
        ],
        out_specs=[
            pl.BlockSpec((_TCB, D), lambda i: (i, 0)),
            pl.BlockSpec((_TCB, D), lambda i: (i, 0)),
        ],
        out_shape=[
            jax.ShapeDtypeStruct((VNUM, D), jnp.float32),
            jax.ShapeDtypeStruct((VNUM, D), jnp.float32),
        ],
    )(partials, loc_W, loc_b, std_W, std_b)


def kernel(sidx, tidx, enorm, esgn, vrepr, loc_W, loc_b, std_W, std_b):
    pad = EP - E
    si2 = jnp.pad(sidx.astype(jnp.int32), (0, pad)).reshape(ROWS, CHUNK)
    ti2 = jnp.pad(tidx.astype(jnp.int32), (0, pad)).reshape(ROWS, CHUNK)
    en2 = jnp.pad(enorm, (0, pad)).reshape(ROWS, CHUNK)
    es2 = jnp.pad(esgn, (0, pad)).reshape(ROWS, CHUNK)

    partials = _sc_graphconv(si2, ti2, en2, es2, vrepr)
    loc, std = _tc_project(partials, loc_W, loc_b.reshape(1, D),
                           std_W, std_b.reshape(1, D))
    return (loc, std)


# speedup vs baseline: 3.4197x; 3.4197x over previous
import dataclasses
import functools

import jax
import jax.numpy as jnp
from jax import lax
from jax.experimental import pallas as pl
from jax.experimental.pallas import tpu as pltpu
from jax.experimental.pallas import tpu_sc as plsc

VNUM = 10000
E = 320000
D = 128
EPS = 1e-7

NC = 2
NS = 16
L = 16
NW = NC * NS

CHUNK = 128
CPW = -(-E // (NW * CHUNK))
ROWS = NW * CPW
EP = ROWS * CHUNK

ZC = 80
NZ = VNUM // ZC

_mesh = plsc.VectorSubcoreMesh(core_axis_name="c", subcore_axis_name="s")

_cp = pltpu.CompilerParams()
if "needs_layout_passes" in pltpu.CompilerParams.__dataclass_fields__:
    _cp = dataclasses.replace(_cp, needs_layout_passes=False)


@functools.partial(
    pl.kernel,
    out_type=jax.ShapeDtypeStruct((NC, VNUM, D), jnp.float32),
    mesh=_mesh,
    scratch_types=[
        pltpu.VMEM_SHARED((VNUM, D), jnp.float32),
        pltpu.VMEM((CHUNK,), jnp.int32),
        pltpu.VMEM((CHUNK,), jnp.int32),
        pltpu.VMEM((CHUNK,), jnp.float32),
        pltpu.VMEM((CHUNK,), jnp.float32),
        pltpu.VMEM((CHUNK, D), jnp.float32),
    ],
    compiler_params=_cp,
)
def _sc_graphconv(sidx_hbm, tidx_hbm, enorm_hbm, esgn_hbm, vrepr_hbm,
                  out_hbm, acc, si_v, ti_v, w_v, e_v, rows_v):
    c = lax.axis_index("c")
    s = lax.axis_index("s")
    wid = s * NC + c

    @pl.loop(0, ZC)
    def _(i):
        @pl.loop(0, D, step=L)
        def _(j):
            rows_v[i, pl.ds(j, L)] = jnp.zeros((L,), jnp.float32)

    @pl.loop(0, (NZ + NS - 1) // NS)
    def _(k):
        j = k * NS + s

        @pl.when(j < NZ)
        def _():
            pltpu.sync_copy(rows_v.at[pl.ds(0, ZC)],
                            acc.at[pl.ds(j * ZC, ZC)])

    plsc.subcore_barrier()

    @pl.loop(0, CPW)
    def _(i):
        row = wid * CPW + i
        pltpu.sync_copy(sidx_hbm.at[row], si_v)
        pltpu.sync_copy(tidx_hbm.at[row], ti_v)
        pltpu.sync_copy(enorm_hbm.at[row], w_v)
        pltpu.sync_copy(esgn_hbm.at[row], e_v)
        pltpu.sync_copy(vrepr_hbm.at[si_v], rows_v)

        @pl.loop(0, CHUNK, step=L)
        def _(j):
            w_v[pl.ds(j, L)] = w_v[pl.ds(j, L)] * e_v[pl.ds(j, L)]

        @pl.loop(0, CHUNK)
        def _(e):
            wv = plsc.load_gather(w_v, [jnp.full((L,), e, jnp.int32)])

            @pl.loop(0, D, step=L)
            def _(j):
                rows_v[e, pl.ds(j, L)] = rows_v[e, pl.ds(j, L)] * wv

        pltpu.sync_copy(rows_v, acc.at[ti_v], add=True)

    plsc.subcore_barrier()

    @pl.loop(0, (NZ + NS - 1) // NS)
    def _(k):
        j = k * NS + s

        @pl.when(j < NZ)
        def _():
            pltpu.sync_copy(acc.at[pl.ds(j * ZC, ZC)],
                            out_hbm.at[c, pl.ds(j * ZC, ZC)])


def _tc_body(p_ref, lw_ref, lb_ref, sw_ref, sb_ref, loc_ref, std_ref):
    ptr = p_ref[0] + p_ref[1]
    dn = (((1,), (1,)), ((), ()))
    loc = lax.dot_general(ptr, lw_ref[...], dn,
                          preferred_element_type=jnp.float32,
                          precision=lax.Precision.HIGHEST)
    loc_ref[...] = loc + lb_ref[...]
    pre = lax.dot_general(ptr, sw_ref[...], dn,
                          preferred_element_type=jnp.float32,
                          precision=lax.Precision.HIGHEST)
    std_ref[...] = jax.nn.softplus(pre + sb_ref[...]) + EPS


_TCB = 1000


def _tc_project(partials, loc_W, loc_b, std_W, std_b):
    grid = (VNUM // _TCB,)
    return pl.pallas_call(
        _tc_body,
        grid=grid,
        in_specs=[
            pl.BlockSpec((NC, _TCB, D), lambda i: (0, i, 0)),
            pl.BlockSpec((D, D), lambda i: (0, 0)),
            pl.BlockSpec((1, D), lambda i: (0, 0)),
            pl.BlockSpec((D, D), lambda i: (0, 0)),
            pl.BlockSpec((1, D), lambda i: (0, 0)),
        ],
        out_specs=[
            pl.BlockSpec((_TCB, D), lambda i: (i, 0)),
            pl.BlockSpec((_TCB, D), lambda i: (i, 0)),
        ],
        out_shape=[
            jax.ShapeDtypeStruct((VNUM, D), jnp.float32),
            jax.ShapeDtypeStruct((VNUM, D), jnp.float32),
        ],
    )(partials, loc_W, loc_b, std_W, std_b)


def kernel(sidx, tidx, enorm, esgn, vrepr, loc_W, loc_b, std_W, std_b):
    pad = EP - E
    si2 = jnp.pad(sidx.astype(jnp.int32), (0, pad)).reshape(ROWS, CHUNK)
    ti2 = jnp.pad(tidx.astype(jnp.int32), (0, pad)).reshape(ROWS, CHUNK)
    en2 = jnp.pad(enorm, (0, pad)).reshape(ROWS, CHUNK)
    es2 = jnp.pad(esgn, (0, pad)).reshape(ROWS, CHUNK)

    partials = _sc_graphconv(si2, ti2, en2, es2, vrepr)
    loc, std = _tc_project(partials, loc_W, loc_b.reshape(1, D),
                           std_W, std_b.reshape(1, D))
    return (loc, std)


# speedup vs baseline: 3.4711x; 1.0150x over previous
import dataclasses
import functools

import jax
import jax.numpy as jnp
from jax import lax
from jax.experimental import pallas as pl
from jax.experimental.pallas import tpu as pltpu
from jax.experimental.pallas import tpu_sc as plsc

VNUM = 10000
E = 320000
D = 128
EPS = 1e-7

NC = 2
NS = 16
L = 16
NW = NC * NS

CHUNK = 128
CPW = 80
ROWS = NW * CPW
EP = ROWS * CHUNK

ZC = 80
NZ = VNUM // ZC

_mesh = plsc.VectorSubcoreMesh(core_axis_name="c", subcore_axis_name="s")

_cp = pltpu.CompilerParams()
if "needs_layout_passes" in pltpu.CompilerParams.__dataclass_fields__:
    _cp = dataclasses.replace(_cp, needs_layout_passes=False)


@functools.partial(
    pl.kernel,
    out_type=jax.ShapeDtypeStruct((NC, VNUM, D), jnp.float32),
    mesh=_mesh,
    scratch_types=[
        pltpu.VMEM_SHARED((VNUM, D), jnp.float32),
        pltpu.VMEM((4, CHUNK), jnp.int32),
        pltpu.VMEM((4, CHUNK), jnp.int32),
        pltpu.VMEM((CHUNK,), jnp.float32),
        pltpu.VMEM((CHUNK, D), jnp.float32),
        pltpu.VMEM((CHUNK, D), jnp.float32),
        pltpu.SemaphoreType.DMA,
        pltpu.SemaphoreType.DMA,
    ],
    compiler_params=_cp,
)
def _sc_graphconv(meta_hbm, vrepr_hbm, out_hbm, acc,
                  m0, m1, w_v, r0, r1, sem0, sem1):
    c = lax.axis_index("c")
    s = lax.axis_index("s")
    wid = s * NC + c
    rows_v = r0

    @pl.loop(0, ZC)
    def _(i):
        @pl.loop(0, D, step=L)
        def _(j):
            rows_v[i, pl.ds(j, L)] = jnp.zeros((L,), jnp.float32)

    @pl.loop(0, (NZ + NS - 1) // NS)
    def _(k):
        j = k * NS + s

        @pl.when(j < NZ)
        def _():
            pltpu.sync_copy(rows_v.at[pl.ds(0, ZC)],
                            acc.at[pl.ds(j * ZC, ZC)])

    plsc.subcore_barrier()

    def _scale_scatter(m, r):
        @pl.loop(0, CHUNK, step=L)
        def _(j):
            en = plsc.bitcast(m[2, pl.ds(j, L)], jnp.float32)
            es = plsc.bitcast(m[3, pl.ds(j, L)], jnp.float32)
            w_v[pl.ds(j, L)] = en * es

        @pl.loop(0, CHUNK)
        def _(e):
            wv = plsc.load_gather(w_v, [jnp.full((L,), e, jnp.int32)])

            @pl.loop(0, D, step=L)
            def _(j):
                r[e, pl.ds(j, L)] = r[e, pl.ds(j, L)] * wv

        pltpu.sync_copy(r, acc.at[m.at[1]], add=True)

    @pl.loop(0, CPW // 2)
    def _(k):
        row = wid * CPW + 2 * k
        pltpu.sync_copy(meta_hbm.at[row], m0)
        d0 = pltpu.async_copy(vrepr_hbm.at[m0.at[0]], r0, sem0)
        pltpu.sync_copy(meta_hbm.at[row + 1], m1)
        d1 = pltpu.async_copy(vrepr_hbm.at[m1.at[0]], r1, sem1)
        d0.wait()
        _scale_scatter(m0, r0)
        d1.wait()
        _scale_scatter(m1, r1)

    plsc.subcore_barrier()

    @pl.loop(0, (NZ + NS - 1) // NS)
    def _(k):
        j = k * NS + s

        @pl.when(j < NZ)
        def _():
            pltpu.sync_copy(acc.at[pl.ds(j * ZC, ZC)],
                            out_hbm.at[c, pl.ds(j * ZC, ZC)])


def _tc_body(p_ref, lw_ref, lb_ref, sw_ref, sb_ref, loc_ref, std_ref):
    ptr = p_ref[0] + p_ref[1]
    dn = (((1,), (1,)), ((), ()))
    loc = lax.dot_general(ptr, lw_ref[...], dn,
                          preferred_element_type=jnp.float32,
                          precision=lax.Precision.HIGHEST)
    loc_ref[...] = loc + lb_ref[...]
    pre = lax.dot_general(ptr, sw_ref[...], dn,
                          preferred_element_type=jnp.float32,
                          precision=lax.Precision.HIGHEST)
    std_ref[...] = jax.nn.softplus(pre + sb_ref[...]) + EPS


_TCB = 1000


def _tc_project(partials, loc_W, loc_b, std_W, std_b):
    grid = (VNUM // _TCB,)
    return pl.pallas_call(
        _tc_body,
        grid=grid,
        in_specs=[
            pl.BlockSpec((NC, _TCB, D), lambda i: (0, i, 0)),
            pl.BlockSpec((D, D), lambda i: (0, 0)),
            pl.BlockSpec((1, D), lambda i: (0, 0)),
            pl.BlockSpec((D, D), lambda i: (0, 0)),
            pl.BlockSpec((1, D), lambda i: (0, 0)),
        ],
        out_specs=[
            pl.BlockSpec((_TCB, D), lambda i: (i, 0)),
            pl.BlockSpec((_TCB, D), lambda i: (i, 0)),
        ],
        out_shape=[
            jax.ShapeDtypeStruct((VNUM, D), jnp.float32),
            jax.ShapeDtypeStruct((VNUM, D), jnp.float32),
        ],
    )(partials, loc_W, loc_b, std_W, std_b)


def kernel(sidx, tidx, enorm, esgn, vrepr, loc_W, loc_b, std_W, std_b):
    pad = EP - E
    si2 = jnp.pad(sidx.astype(jnp.int32), (0, pad)).reshape(ROWS, CHUNK)
    ti2 = jnp.pad(tidx.astype(jnp.int32), (0, pad)).reshape(ROWS, CHUNK)
    en2 = lax.bitcast_convert_type(
        jnp.pad(enorm, (0, pad)), jnp.int32).reshape(ROWS, CHUNK)
    es2 = lax.bitcast_convert_type(
        jnp.pad(esgn, (0, pad)), jnp.int32).reshape(ROWS, CHUNK)
    meta = jnp.stack([si2, ti2, en2, es2], axis=1)

    partials = _sc_graphconv(meta, vrepr)
    loc, std = _tc_project(partials, loc_W, loc_b.reshape(1, D),
                           std_W, std_b.reshape(1, D))
    return (loc, std)
